# baseline (speedup 1.0000x reference)
import functools

import jax
import jax.numpy as jnp
from jax import lax
from jax.experimental import pallas as pl
from jax.experimental.pallas import tpu as pltpu
from jax.experimental.pallas import tpu_sc as plsc

VOCAB = 1000000
CONTENT_DIM = 64
TIME_DIM = 16
OUT_DIM = CONTENT_DIM + TIME_DIM
BATCH = 4096
SEQ = 200

_INFO = plsc.get_sparse_core_info()
NC, NS = _INFO.num_cores, _INFO.num_subcores
NW = NC * NS

CHUNK = 128
BT = BATCH // CHUNK
DT = OUT_DIM // 8
N_CHUNKS = SEQ * BT
CH_PER_W = N_CHUNKS // NW


def _gather_body(ids_hbm, content_hbm, time_hbm, out_hbm,
                 idx_v, rows_c, rows_t, qbuf, sem_c, sem_t):
    wid = lax.axis_index("s") * NC + lax.axis_index("c")
    lane = lax.iota(jnp.int32, 16)

    def chunk_body(i, carry):
        c = wid * CH_PER_W + i
        s = c // BT
        bt = c % BT
        n0 = s * BATCH + bt * CHUNK
        pltpu.sync_copy(ids_hbm.at[pl.ds(n0, CHUNK)], idx_v)
        cp_c = pltpu.async_copy(content_hbm.at[idx_v], rows_c, sem_c)
        cp_t = pltpu.async_copy(time_hbm.at[idx_v], rows_t, sem_t)
        cp_c.wait()
        cp_t.wait()

        def tr_body(j, carry2):
            jv = jnp.full((16,), j, jnp.int32)
            for k in range(CONTENT_DIM // 16):
                x = rows_c[j, pl.ds(k * 16, 16)]
                plsc.store_scatter(qbuf, [k * 16 + lane, jv], x)
            x = rows_t[j, pl.ds(0, 16)]
            plsc.store_scatter(qbuf, [CONTENT_DIM + lane, jv], x)
            return carry2

        lax.fori_loop(0, CHUNK, tr_body, 0, unroll=2)

        for dt in range(DT):
            pltpu.sync_copy(qbuf.at[pl.ds(dt * 8, 8), :], out_hbm.at[s, dt, bt])
        return carry

    lax.fori_loop(0, CH_PER_W, chunk_body, 0)


@jax.jit
def kernel(token_ids, content_table, time_table):
    ids = token_ids.T.reshape(BATCH * SEQ)

    mesh = plsc.VectorSubcoreMesh(core_axis_name="c", subcore_axis_name="s")
    q = pl.kernel(
        _gather_body,
        out_type=jax.ShapeDtypeStruct((SEQ, DT, BT, 8, CHUNK), jnp.float32),
        mesh=mesh,
        scratch_types=[
            pltpu.VMEM((CHUNK,), jnp.int32),
            pltpu.VMEM((CHUNK, CONTENT_DIM), jnp.float32),
            pltpu.VMEM((CHUNK, TIME_DIM), jnp.float32),
            pltpu.VMEM((OUT_DIM, CHUNK), jnp.float32),
            pltpu.SemaphoreType.DMA,
            pltpu.SemaphoreType.DMA,
        ],
        compiler_params=pltpu.CompilerParams(
            use_tc_tiling_on_sc=False, needs_layout_passes=False),
    )(ids, content_table, time_table)
    return q.transpose(2, 4, 0, 1, 3).reshape(BATCH, SEQ, OUT_DIM)
